# baseline (speedup 1.0000x reference)
import functools

import jax
import jax.numpy as jnp
from jax import lax
from jax.experimental import pallas as pl
from jax.experimental.pallas import tpu as pltpu
from jax.experimental.pallas import tpu_sc as plsc

N = 10000
NP = 10240
E = 320000
DIN = 128
H = 64
R = 3

NC = 2
NS = 16
NW = NC * NS
EW = E // NW
CK = 80
NCHUNK = EW // CK
CKS = 80
NCHUNKS = EW // CKS
ROWS = R * N
ROWS_PAD = R * NP
ROWS_T = ROWS_PAD // NS
CNT_PAD = 30720
CNT_T = CNT_PAD // NS

_MESH = plsc.VectorSubcoreMesh(core_axis_name="core", subcore_axis_name="subcore")
_SC_PARAMS = pltpu.CompilerParams(use_tc_tiling_on_sc=False)


def _idx_body(et_ref, src_ref, dst_ref, g_ref, s_ref):
    et = et_ref[...]
    g_ref[...] = et * N + src_ref[...]
    s_ref[...] = et * NP + dst_ref[...]


_idx_call = pl.pallas_call(
    _idx_body,
    out_shape=(
        jax.ShapeDtypeStruct((E // 128, 128), jnp.int32),
        jax.ShapeDtypeStruct((E // 128, 128), jnp.int32),
    ),
)


BM = 1000
NB = N // BM


def _mm4_body(x_ref, w_ref, root_ref, ytab_ref):
    m = jnp.dot(x_ref[...].astype(jnp.bfloat16), w_ref[0],
                preferred_element_type=jnp.float32)
    root_ref[...] = m
    ytab_ref[...] = m.astype(jnp.bfloat16)


def _make_mm4(din):
    return pl.pallas_call(
        _mm4_body,
        grid=(NB, R + 1),
        in_specs=[
            pl.BlockSpec((BM, din), lambda i, r: (i, 0)),
            pl.BlockSpec((1, din, H), lambda i, r: (r, 0, 0)),
        ],
        out_specs=(
            pl.BlockSpec((BM, H), lambda i, r: (jnp.where(r == 0, i, NB), 0)),
            pl.BlockSpec((BM, H),
                         lambda i, r: (jnp.where(r == 0, R * NB,
                                                 (r - 1) * NB + i), 0)),
        ),
        out_shape=(
            jax.ShapeDtypeStruct((N + BM, H), jnp.float32),
            jax.ShapeDtypeStruct((ROWS + BM, H), jnp.bfloat16),
        ),
    )


_mm4_din = _make_mm4(DIN)
_mm4_h = _make_mm4(H)


RING = 5
MAIN_ROUNDS = NCHUNK // RING
RINGS = 5
ROUNDS_S = NCHUNKS // RINGS


def _scatter_body(with_counts):
    def body(ytab_h, gidx_h, sidx_h, zrows_h, *rest):
        if with_counts:
            zcnt_h, ones_h, acc_out, cnt_out, gbuf, sbuf, msg, ones_v, \
                acc_sh, cnt_sh = rest[:10]
            sems = rest[10:]
        else:
            acc_out, gbuf, sbuf, msg, acc_sh = rest[:5]
            sems = rest[5:]
        gsem = sems[:RINGS]
        ssem = sems[RINGS:2 * RINGS]
        c = lax.axis_index("core")
        s = lax.axis_index("subcore")
        wid = c * NS + s
        pltpu.sync_copy(zrows_h, acc_sh.at[pl.ds(s * ROWS_T, ROWS_T)])
        if with_counts:
            pltpu.sync_copy(zcnt_h, cnt_sh.at[pl.ds(s * CNT_T, CNT_T)])
            pltpu.sync_copy(ones_h, ones_v)
        plsc.subcore_barrier()

        def load_and_gather(b, ch):
            base = wid * EW + ch * CKS
            pltpu.sync_copy(gidx_h.at[pl.ds(base, CKS)], gbuf.at[b])
            pltpu.sync_copy(sidx_h.at[pl.ds(base, CKS)], sbuf.at[b])
            pltpu.async_copy(ytab_h.at[gbuf.at[b]], msg.at[b], gsem[b])

        def gather_wait(b):
            pltpu.make_async_copy(ytab_h.at[gbuf.at[b]], msg.at[b],
                                  gsem[b]).wait()

        def scatter_wait(b):
            pltpu.make_async_copy(msg.at[b], acc_sh.at[sbuf.at[b]],
                                  ssem[b]).wait()

        for b in range(RINGS):
            load_and_gather(b, b)

        @pl.loop(0, ROUNDS_S)
        def _(k):
            for b in range(RINGS):
                ch = RINGS * k + b
                gather_wait(b)
                pltpu.async_copy(msg.at[b], acc_sh.at[sbuf.at[b]], ssem[b],
                                 add=True)
                if with_counts:
                    pltpu.sync_copy(ones_v, cnt_sh.at[sbuf.at[b]], add=True)

                @pl.when(k < ROUNDS_S - 1)
                def _():
                    scatter_wait(b)
                    load_and_gather(b, ch + RINGS)

        for b in range(RINGS):
            scatter_wait(b)

        plsc.subcore_barrier()
        pltpu.sync_copy(acc_sh.at[pl.ds(s * ROWS_T, ROWS_T)],
                        acc_out.at[c, pl.ds(s * ROWS_T, ROWS_T)])
        if with_counts:
            pltpu.sync_copy(cnt_sh.at[pl.ds(s * CNT_T, CNT_T)],
                            cnt_out.at[c, pl.ds(s * CNT_T, CNT_T)])

    return body


_SEM_RING = [pltpu.SemaphoreType.DMA] * (2 * RING)
_SEM_RING_S = [pltpu.SemaphoreType.DMA] * (2 * RINGS)


def _sc_scatter_counts(ytab, gidx, sidx, zrows, zcnt, ones_ck):
    f = pl.kernel(
        _scatter_body(True),
        out_type=(
            jax.ShapeDtypeStruct((NC, ROWS_PAD, H), jnp.bfloat16),
            jax.ShapeDtypeStruct((NC, CNT_PAD), jnp.float32),
        ),
        mesh=_MESH,
        compiler_params=_SC_PARAMS,
        scratch_types=[
            pltpu.VMEM((RINGS, CKS), jnp.int32),
            pltpu.VMEM((RINGS, CKS), jnp.int32),
            pltpu.VMEM((RINGS, CKS, H), jnp.bfloat16),
            pltpu.VMEM((CKS,), jnp.float32),
            pltpu.VMEM_SHARED((ROWS_PAD, H), jnp.bfloat16),
            pltpu.VMEM_SHARED((CNT_PAD,), jnp.float32),
        ] + _SEM_RING_S,
    )
    return f(ytab, gidx, sidx, zrows, zcnt, ones_ck)


def _sc_scatter(ytab, gidx, sidx, zrows):
    f = pl.kernel(
        _scatter_body(False),
        out_type=jax.ShapeDtypeStruct((NC, ROWS_PAD, H), jnp.bfloat16),
        mesh=_MESH,
        compiler_params=_SC_PARAMS,
        scratch_types=[
            pltpu.VMEM((RINGS, CKS), jnp.int32),
            pltpu.VMEM((RINGS, CKS), jnp.int32),
            pltpu.VMEM((RINGS, CKS, H), jnp.bfloat16),
            pltpu.VMEM_SHARED((ROWS_PAD, H), jnp.bfloat16),
        ] + _SEM_RING_S,
    )
    return f(ytab, gidx, sidx, zrows)


def _combine_body(root, a0, a1, a2, b0, b1, b2, c0, c1, c2, d0, d1, d2,
                  bias, o):
    h = root[...] + bias[...]
    for aa, bb, cc, dd in ((a0, b0, c0, d0), (a1, b1, c1, d1),
                           (a2, b2, c2, d2)):
        cnt = jnp.maximum(cc[...] + dd[...], 1.0)
        h = h + (aa[...].astype(jnp.float32)
                 + bb[...].astype(jnp.float32)) / cnt
    o[...] = jnp.maximum(h, 0.0)


def _make_combine(bm=1000):
    nh_spec = pl.BlockSpec((bm, H), lambda i: (i, 0))
    n1_spec = pl.BlockSpec((bm, 1), lambda i: (i, 0))
    b_spec = pl.BlockSpec((1, H), lambda i: (0, 0))
    return pl.pallas_call(
        _combine_body,
        grid=(N // bm,),
        in_specs=[nh_spec] * 7 + [n1_spec] * 6 + [b_spec],
        out_specs=nh_spec,
        out_shape=jax.ShapeDtypeStruct((N, H), jnp.float32),
    )


_combine = _make_combine()


def _combine3_body(root, a0, a1, a2, b0, b1, b2, c0, c1, c2, d0, d1, d2,
                   bias, g, bln, wa, wb, nw1, nb1, nw2, nb2,
                   p_o, q_o, node_o):
    h = root[...] + bias[...]
    for aa, bb, cc, dd in ((a0, b0, c0, d0), (a1, b1, c1, d1),
                           (a2, b2, c2, d2)):
        cnt = jnp.maximum(cc[...] + dd[...], 1.0)
        h = h + (aa[...].astype(jnp.float32)
                 + bb[...].astype(jnp.float32)) / cnt
    mu = jnp.mean(h, axis=-1, keepdims=True)
    var = jnp.mean((h - mu) ** 2, axis=-1, keepdims=True)
    xn = g[...] * (h - mu) / jnp.sqrt(var + 1e-5) + bln[...]
    p_o[...] = jnp.dot(xn, wa[...], preferred_element_type=jnp.float32)
    q_o[...] = jnp.dot(xn, wb[...], preferred_element_type=jnp.float32)
    nh = jnp.maximum(
        jnp.dot(xn, nw1[...], preferred_element_type=jnp.float32) + nb1[...],
        0.0)
    lg = jnp.dot(nh, nw2[...], preferred_element_type=jnp.float32) + nb2[...]
    m = jnp.max(lg, axis=-1, keepdims=True)
    l = lg - m
    node_o[...] = l - jnp.log(jnp.sum(jnp.exp(l), axis=-1, keepdims=True))


def _make_combine3(bm=1000):
    nh_spec = pl.BlockSpec((bm, H), lambda i: (i, 0))
    n1_spec = pl.BlockSpec((bm, 1), lambda i: (i, 0))
    b_spec = pl.BlockSpec((1, H), lambda i: (0, 0))
    return pl.pallas_call(
        _combine3_body,
        grid=(N // bm,),
        in_specs=(
            [nh_spec] * 7 + [n1_spec] * 6 + [b_spec] * 3
            + [pl.BlockSpec((H, H), lambda i: (0, 0))] * 2
            + [pl.BlockSpec((H, H // 2), lambda i: (0, 0)),
               pl.BlockSpec((1, H // 2), lambda i: (0, 0)),
               pl.BlockSpec((H // 2, 2), lambda i: (0, 0)),
               pl.BlockSpec((1, 2), lambda i: (0, 0))]
        ),
        out_specs=(
            nh_spec,
            nh_spec,
            pl.BlockSpec((bm, 2), lambda i: (i, 0)),
        ),
        out_shape=(
            jax.ShapeDtypeStruct((N, H), jnp.float32),
            jax.ShapeDtypeStruct((N, H), jnp.float32),
            jax.ShapeDtypeStruct((N, 2), jnp.float32),
        ),
    )


_combine3 = _make_combine3()


E2 = E // 2
EW2 = E2 // NW
CK2 = 40
NCHUNK2 = EW2 // CK2
ROUNDS_E = NCHUNK2 // RING


def _sc_edge_gather(p, q, src, dst):
    def body(p_h, q_h, src_h, dst_h, ehpq_out, sibuf, dibuf, bp, bq, *sems):
        gsem = sems[:RING]
        wsem = sems[RING:2 * RING]
        c = lax.axis_index("core")
        s = lax.axis_index("subcore")
        wid = c * NS + s

        def load_and_gather(b, ch):
            base = wid * EW2 + ch * CK2
            pltpu.sync_copy(src_h.at[pl.ds(base, CK2)], sibuf.at[b])
            pltpu.sync_copy(dst_h.at[pl.ds(base, CK2)], dibuf.at[b])
            pltpu.async_copy(p_h.at[sibuf.at[b]], bp.at[b], gsem[b])
            pltpu.async_copy(q_h.at[dibuf.at[b]], bq.at[b], gsem[b])

        def gather_wait(b):
            pltpu.make_async_copy(p_h.at[sibuf.at[b]], bp.at[b],
                                  gsem[b]).wait()
            pltpu.make_async_copy(q_h.at[dibuf.at[b]], bq.at[b],
                                  gsem[b]).wait()

        def write_start(b, ch):
            base = wid * EW2 + ch * CK2
            pltpu.async_copy(bp.at[b],
                             ehpq_out.at[pl.ds(base, CK2), pl.ds(0, H)],
                             wsem[b])
            pltpu.async_copy(bq.at[b],
                             ehpq_out.at[pl.ds(base, CK2), pl.ds(H, H)],
                             wsem[b])

        def write_wait(b, ch):
            base = wid * EW2 + ch * CK2
            pltpu.make_async_copy(bp.at[b],
                                  ehpq_out.at[pl.ds(base, CK2), pl.ds(0, H)],
                                  wsem[b]).wait()
            pltpu.make_async_copy(bq.at[b],
                                  ehpq_out.at[pl.ds(base, CK2), pl.ds(H, H)],
                                  wsem[b]).wait()

        for b in range(RING):
            load_and_gather(b, b)

        @pl.loop(0, ROUNDS_E)
        def _(k):
            for b in range(RING):
                ch = RING * k + b
                gather_wait(b)
                write_start(b, ch)

                @pl.when(k < ROUNDS_E - 1)
                def _():
                    write_wait(b, ch)
                    load_and_gather(b, ch + RING)

        for b in range(RING):
            write_wait(b, 0)

    f = pl.kernel(
        body,
        out_type=jax.ShapeDtypeStruct((E2, 2 * H), jnp.float32),
        mesh=_MESH,
        compiler_params=_SC_PARAMS,
        scratch_types=[
            pltpu.VMEM((RING, CK2), jnp.int32),
            pltpu.VMEM((RING, CK2), jnp.int32),
            pltpu.VMEM((RING, CK2, H), jnp.float32),
            pltpu.VMEM((RING, CK2, H), jnp.float32),
        ] + _SEM_RING,
    )
    return f(p, q, src, dst)


def _edge_out_body(pq_ref, eb1, w2p8, b2p8, o_ref):
    blk = pq_ref[...]
    eh = jnp.maximum(blk[:, :H] + blk[:, H:] + eb1[...], 0.0)
    logits = lax.dot_general(
        w2p8[...], eh.astype(jnp.bfloat16), (((1,), (1,)), ((), ())),
        preferred_element_type=jnp.float32) + b2p8[...]
    mask = lax.broadcasted_iota(jnp.int32, logits.shape, 0) < 3
    lm = jnp.where(mask, logits, -1e30)
    m = jnp.max(lm, axis=0, keepdims=True)
    ex = jnp.where(mask, jnp.exp(logits - m), 0.0)
    lse = jnp.log(jnp.sum(ex, axis=0, keepdims=True))
    o_ref[...] = logits - m - lse


def _make_edge_out(bm=6400):
    return pl.pallas_call(
        _edge_out_body,
        grid=(E2 // bm,),
        in_specs=[
            pl.BlockSpec((bm, 2 * H), lambda i: (i, 0)),
            pl.BlockSpec((1, H), lambda i: (0, 0)),
            pl.BlockSpec((8, H), lambda i: (0, 0)),
            pl.BlockSpec((8, 1), lambda i: (0, 0)),
        ],
        out_specs=pl.BlockSpec((8, bm), lambda i: (0, i)),
        out_shape=jax.ShapeDtypeStruct((8, E2), jnp.float32),
    )


_edge_out = _make_edge_out()


def _rgcn_layer(x_in, w_rel, w_root, mm4, gidx, sidx, zrows,
                zcnt=None, ones_ck=None, counts=None):
    w_all = jnp.concatenate([w_root[None], w_rel],
                            axis=0).astype(jnp.bfloat16)
    root, ytab = mm4(x_in, w_all)
    if counts is None:
        acc, cnt = _sc_scatter_counts(ytab, gidx, sidx, zrows, zcnt, ones_ck)
        counts = (cnt[0].reshape(R, NP, 1), cnt[1].reshape(R, NP, 1))
    else:
        acc = _sc_scatter(ytab, gidx, sidx, zrows)
    acc_a = acc[0].reshape(R, NP, H)
    acc_b = acc[1].reshape(R, NP, H)
    ca, cb = counts
    parts = ([root] + [acc_a[r] for r in range(R)] + [acc_b[r] for r in range(R)]
             + [ca[r] for r in range(R)] + [cb[r] for r in range(R)])
    return parts, counts


def kernel(x, edge_index, edge_type, w1_rel, w1_root, b1, w2_rel, w2_root, b2,
           w3_rel, w3_root, b3, ln_g, ln_b, ec_w1, ec_b1, ec_w2, ec_b2,
           nc_w1, nc_b1, nc_w2, nc_b2):
    src = edge_index[0]
    dst = edge_index[1]
    g2, s2 = _idx_call(edge_type.reshape(E // 128, 128),
                       src.reshape(E // 128, 128),
                       dst.reshape(E // 128, 128))
    gidx = g2.reshape(E)
    sidx = s2.reshape(E)

    zrows = jnp.zeros((ROWS_T, H), jnp.bfloat16)
    zcnt = jnp.zeros((CNT_T,), jnp.float32)
    ones_ck = jnp.ones((CKS,), jnp.float32)

    parts, counts = _rgcn_layer(x, w1_rel, w1_root, _mm4_din, gidx, sidx,
                                zrows, zcnt=zcnt, ones_ck=ones_ck)
    x1 = _combine(*parts, b1.reshape(1, H))

    parts, _ = _rgcn_layer(x1, w2_rel, w2_root, _mm4_h, gidx, sidx, zrows,
                           counts=counts)
    x2 = _combine(*parts, b2.reshape(1, H))

    parts, _ = _rgcn_layer(x2, w3_rel, w3_root, _mm4_h, gidx, sidx, zrows,
                           counts=counts)
    ec_w1t = ec_w1.T
    p, q, node_out = _combine3(
        *parts, b3.reshape(1, H), ln_g.reshape(1, H), ln_b.reshape(1, H),
        ec_w1t[:H], ec_w1t[H:], nc_w1.T, nc_b1.reshape(1, H // 2),
        nc_w2.T, nc_b2.reshape(1, 2))

    w2p8 = jnp.pad(ec_w2, ((0, 5), (0, 0))).astype(jnp.bfloat16)
    b2p8 = jnp.pad(ec_b2, (0, 5)).reshape(8, 1)
    eb1 = ec_b1.reshape(1, H)
    ehpq_a = _sc_edge_gather(p, q, src[:E2], dst[:E2])
    ehpq_b = _sc_edge_gather(p, q, src[E2:], dst[E2:])
    lt_a = _edge_out(ehpq_a, eb1, w2p8, b2p8)
    lt_b = _edge_out(ehpq_b, eb1, w2p8, b2p8)
    edge_out = jnp.concatenate([lt_a[:3].T, lt_b[:3].T], axis=0)

    return (edge_out, node_out)


# speedup vs baseline: 1.0776x; 1.0776x over previous
import functools

import jax
import jax.numpy as jnp
from jax import lax
from jax.experimental import pallas as pl
from jax.experimental.pallas import tpu as pltpu
from jax.experimental.pallas import tpu_sc as plsc

N = 10000
NP = 10240
E = 320000
DIN = 128
H = 64
R = 3

NC = 2
NS = 16
NW = NC * NS
EW = E // NW
CK = 80
NCHUNK = EW // CK
CKS = 80
NCHUNKS = EW // CKS
ROWS = R * N
ROWS_PAD = R * NP
ROWS_T = ROWS_PAD // NS
CNT_PAD = 30720
CNT_T = CNT_PAD // NS

_MESH = plsc.VectorSubcoreMesh(core_axis_name="core", subcore_axis_name="subcore")
_SC_PARAMS = pltpu.CompilerParams(use_tc_tiling_on_sc=False)


def _idx_body(et_ref, src_ref, dst_ref, g_ref, s_ref):
    et = et_ref[...]
    g_ref[...] = et * N + src_ref[...]
    s_ref[...] = et * NP + dst_ref[...]


_idx_call = pl.pallas_call(
    _idx_body,
    out_shape=(
        jax.ShapeDtypeStruct((E // 128, 128), jnp.int32),
        jax.ShapeDtypeStruct((E // 128, 128), jnp.int32),
    ),
)


BM = 1000
NB = N // BM


def _mm4_body(x_ref, w_ref, root_ref, ytab_ref):
    m = jnp.dot(x_ref[...].astype(jnp.bfloat16), w_ref[0],
                preferred_element_type=jnp.float32)
    root_ref[...] = m
    ytab_ref[...] = m.astype(jnp.bfloat16)


def _make_mm4(din):
    return pl.pallas_call(
        _mm4_body,
        grid=(NB, R + 1),
        in_specs=[
            pl.BlockSpec((BM, din), lambda i, r: (i, 0)),
            pl.BlockSpec((1, din, H), lambda i, r: (r, 0, 0)),
        ],
        out_specs=(
            pl.BlockSpec((BM, H), lambda i, r: (jnp.where(r == 0, i, NB), 0)),
            pl.BlockSpec((BM, H),
                         lambda i, r: (jnp.where(r == 0, R * NB,
                                                 (r - 1) * NB + i), 0)),
        ),
        out_shape=(
            jax.ShapeDtypeStruct((N + BM, H), jnp.float32),
            jax.ShapeDtypeStruct((ROWS + BM, H), jnp.bfloat16),
        ),
    )


_mm4_din = _make_mm4(DIN)
_mm4_h = _make_mm4(H)


RING = 5
MAIN_ROUNDS = NCHUNK // RING
RINGS = 5
ROUNDS_S = NCHUNKS // RINGS


def _scatter_body(with_counts):
    def body(ytab_h, gidx_h, sidx_h, zrows_h, *rest):
        if with_counts:
            zcnt_h, ones_h, acc_out, cnt_out, gbuf, sbuf, msg, ones_v, \
                acc_sh, cnt_sh = rest[:10]
            sems = rest[10:]
        else:
            acc_out, gbuf, sbuf, msg, acc_sh = rest[:5]
            sems = rest[5:]
        gsem = sems[:RINGS]
        ssem = sems[RINGS:2 * RINGS]
        c = lax.axis_index("core")
        s = lax.axis_index("subcore")
        wid = c * NS + s
        pltpu.sync_copy(zrows_h, acc_sh.at[pl.ds(s * ROWS_T, ROWS_T)])
        if with_counts:
            pltpu.sync_copy(zcnt_h, cnt_sh.at[pl.ds(s * CNT_T, CNT_T)])
            pltpu.sync_copy(ones_h, ones_v)
        plsc.subcore_barrier()

        def load_and_gather(b, ch):
            base = wid * EW + ch * CKS
            pltpu.sync_copy(gidx_h.at[pl.ds(base, CKS)], gbuf.at[b])
            pltpu.sync_copy(sidx_h.at[pl.ds(base, CKS)], sbuf.at[b])
            pltpu.async_copy(ytab_h.at[gbuf.at[b]], msg.at[b], gsem[b])

        def gather_wait(b):
            pltpu.make_async_copy(ytab_h.at[gbuf.at[b]], msg.at[b],
                                  gsem[b]).wait()

        def scatter_wait(b):
            pltpu.make_async_copy(msg.at[b], acc_sh.at[sbuf.at[b]],
                                  ssem[b]).wait()

        for b in range(RINGS):
            load_and_gather(b, b)

        @pl.loop(0, ROUNDS_S)
        def _(k):
            for b in range(RINGS):
                ch = RINGS * k + b
                gather_wait(b)
                pltpu.async_copy(msg.at[b], acc_sh.at[sbuf.at[b]], ssem[b],
                                 add=True)
                if with_counts:
                    pltpu.sync_copy(ones_v, cnt_sh.at[sbuf.at[b]], add=True)

                @pl.when(k < ROUNDS_S - 1)
                def _():
                    scatter_wait(b)
                    load_and_gather(b, ch + RINGS)

        for b in range(RINGS):
            scatter_wait(b)

        plsc.subcore_barrier()
        pltpu.sync_copy(acc_sh.at[pl.ds(s * ROWS_T, ROWS_T)],
                        acc_out.at[c, pl.ds(s * ROWS_T, ROWS_T)])
        if with_counts:
            pltpu.sync_copy(cnt_sh.at[pl.ds(s * CNT_T, CNT_T)],
                            cnt_out.at[c, pl.ds(s * CNT_T, CNT_T)])

    return body


_SEM_RING = [pltpu.SemaphoreType.DMA] * (2 * RING)
_SEM_RING_S = [pltpu.SemaphoreType.DMA] * (2 * RINGS)


def _sc_scatter_counts(ytab, gidx, sidx, zrows, zcnt, ones_ck):
    f = pl.kernel(
        _scatter_body(True),
        out_type=(
            jax.ShapeDtypeStruct((NC, ROWS_PAD, H), jnp.bfloat16),
            jax.ShapeDtypeStruct((NC, CNT_PAD), jnp.float32),
        ),
        mesh=_MESH,
        compiler_params=_SC_PARAMS,
        scratch_types=[
            pltpu.VMEM((RINGS, CKS), jnp.int32),
            pltpu.VMEM((RINGS, CKS), jnp.int32),
            pltpu.VMEM((RINGS, CKS, H), jnp.bfloat16),
            pltpu.VMEM((CKS,), jnp.float32),
            pltpu.VMEM_SHARED((ROWS_PAD, H), jnp.bfloat16),
            pltpu.VMEM_SHARED((CNT_PAD,), jnp.float32),
        ] + _SEM_RING_S,
    )
    return f(ytab, gidx, sidx, zrows, zcnt, ones_ck)


def _sc_scatter(ytab, gidx, sidx, zrows):
    f = pl.kernel(
        _scatter_body(False),
        out_type=jax.ShapeDtypeStruct((NC, ROWS_PAD, H), jnp.bfloat16),
        mesh=_MESH,
        compiler_params=_SC_PARAMS,
        scratch_types=[
            pltpu.VMEM((RINGS, CKS), jnp.int32),
            pltpu.VMEM((RINGS, CKS), jnp.int32),
            pltpu.VMEM((RINGS, CKS, H), jnp.bfloat16),
            pltpu.VMEM_SHARED((ROWS_PAD, H), jnp.bfloat16),
        ] + _SEM_RING_S,
    )
    return f(ytab, gidx, sidx, zrows)


def _combine_body(root, a0, a1, a2, b0, b1, b2, c0, c1, c2, d0, d1, d2,
                  bias, o):
    h = root[...] + bias[...]
    for aa, bb, cc, dd in ((a0, b0, c0, d0), (a1, b1, c1, d1),
                           (a2, b2, c2, d2)):
        cnt = jnp.maximum(cc[...] + dd[...], 1.0)
        h = h + (aa[...].astype(jnp.float32)
                 + bb[...].astype(jnp.float32)) / cnt
    o[...] = jnp.maximum(h, 0.0)


def _make_combine(bm=1000):
    nh_spec = pl.BlockSpec((bm, H), lambda i: (i, 0))
    n1_spec = pl.BlockSpec((bm, 1), lambda i: (i, 0))
    b_spec = pl.BlockSpec((1, H), lambda i: (0, 0))
    return pl.pallas_call(
        _combine_body,
        grid=(N // bm,),
        in_specs=[nh_spec] * 7 + [n1_spec] * 6 + [b_spec],
        out_specs=nh_spec,
        out_shape=jax.ShapeDtypeStruct((N, H), jnp.float32),
    )


_combine = _make_combine()


def _combine3_body(root, a0, a1, a2, b0, b1, b2, c0, c1, c2, d0, d1, d2,
                   bias, g, bln, wa, wb, nw1, nb1, nw2, nb2,
                   p_o, q_o, node_o):
    h = root[...] + bias[...]
    for aa, bb, cc, dd in ((a0, b0, c0, d0), (a1, b1, c1, d1),
                           (a2, b2, c2, d2)):
        cnt = jnp.maximum(cc[...] + dd[...], 1.0)
        h = h + (aa[...].astype(jnp.float32)
                 + bb[...].astype(jnp.float32)) / cnt
    mu = jnp.mean(h, axis=-1, keepdims=True)
    var = jnp.mean((h - mu) ** 2, axis=-1, keepdims=True)
    xn = g[...] * (h - mu) / jnp.sqrt(var + 1e-5) + bln[...]
    p_o[...] = jnp.dot(xn, wa[...], preferred_element_type=jnp.float32)
    q_o[...] = jnp.dot(xn, wb[...], preferred_element_type=jnp.float32)
    nh = jnp.maximum(
        jnp.dot(xn, nw1[...], preferred_element_type=jnp.float32) + nb1[...],
        0.0)
    lg = jnp.dot(nh, nw2[...], preferred_element_type=jnp.float32) + nb2[...]
    m = jnp.max(lg, axis=-1, keepdims=True)
    l = lg - m
    node_o[...] = l - jnp.log(jnp.sum(jnp.exp(l), axis=-1, keepdims=True))


def _make_combine3(bm=1000):
    nh_spec = pl.BlockSpec((bm, H), lambda i: (i, 0))
    n1_spec = pl.BlockSpec((bm, 1), lambda i: (i, 0))
    b_spec = pl.BlockSpec((1, H), lambda i: (0, 0))
    return pl.pallas_call(
        _combine3_body,
        grid=(N // bm,),
        in_specs=(
            [nh_spec] * 7 + [n1_spec] * 6 + [b_spec] * 3
            + [pl.BlockSpec((H, H), lambda i: (0, 0))] * 2
            + [pl.BlockSpec((H, H // 2), lambda i: (0, 0)),
               pl.BlockSpec((1, H // 2), lambda i: (0, 0)),
               pl.BlockSpec((H // 2, 2), lambda i: (0, 0)),
               pl.BlockSpec((1, 2), lambda i: (0, 0))]
        ),
        out_specs=(
            nh_spec,
            nh_spec,
            pl.BlockSpec((bm, 2), lambda i: (i, 0)),
        ),
        out_shape=(
            jax.ShapeDtypeStruct((N, H), jnp.float32),
            jax.ShapeDtypeStruct((N, H), jnp.float32),
            jax.ShapeDtypeStruct((N, 2), jnp.float32),
        ),
    )


_combine3 = _make_combine3()


EA = 192000
EB = E - EA


def _sc_edge_gather(p, q, src, dst, ne):
    ew2 = ne // NW
    nchunk2 = ew2 // CK
    rounds_e = nchunk2 // RING
    def body(p_h, q_h, src_h, dst_h, ehpq_out, sibuf, dibuf, bp, bq, *sems):
        gsem = sems[:RING]
        wsem = sems[RING:2 * RING]
        c = lax.axis_index("core")
        s = lax.axis_index("subcore")
        wid = c * NS + s

        def load_and_gather(b, ch):
            base = wid * ew2 + ch * CK
            pltpu.sync_copy(src_h.at[pl.ds(base, CK)], sibuf.at[b])
            pltpu.sync_copy(dst_h.at[pl.ds(base, CK)], dibuf.at[b])
            pltpu.async_copy(p_h.at[sibuf.at[b]], bp.at[b], gsem[b])
            pltpu.async_copy(q_h.at[dibuf.at[b]], bq.at[b], gsem[b])

        def gather_wait(b):
            pltpu.make_async_copy(p_h.at[sibuf.at[b]], bp.at[b],
                                  gsem[b]).wait()
            pltpu.make_async_copy(q_h.at[dibuf.at[b]], bq.at[b],
                                  gsem[b]).wait()

        def write_start(b, ch):
            base = wid * ew2 + ch * CK
            pltpu.async_copy(bp.at[b],
                             ehpq_out.at[pl.ds(base, CK), pl.ds(0, H)],
                             wsem[b])
            pltpu.async_copy(bq.at[b],
                             ehpq_out.at[pl.ds(base, CK), pl.ds(H, H)],
                             wsem[b])

        def write_wait(b, ch):
            base = wid * ew2 + ch * CK
            pltpu.make_async_copy(bp.at[b],
                                  ehpq_out.at[pl.ds(base, CK), pl.ds(0, H)],
                                  wsem[b]).wait()
            pltpu.make_async_copy(bq.at[b],
                                  ehpq_out.at[pl.ds(base, CK), pl.ds(H, H)],
                                  wsem[b]).wait()

        for b in range(RING):
            load_and_gather(b, b)

        @pl.loop(0, rounds_e)
        def _(k):
            for b in range(RING):
                ch = RING * k + b
                gather_wait(b)
                write_start(b, ch)

                @pl.when(k < rounds_e - 1)
                def _():
                    write_wait(b, ch)
                    load_and_gather(b, ch + RING)

        for b in range(RING):
            write_wait(b, 0)

    f = pl.kernel(
        body,
        out_type=jax.ShapeDtypeStruct((ne, 2 * H), jnp.float32),
        mesh=_MESH,
        compiler_params=_SC_PARAMS,
        scratch_types=[
            pltpu.VMEM((RING, CK), jnp.int32),
            pltpu.VMEM((RING, CK), jnp.int32),
            pltpu.VMEM((RING, CK, H), jnp.float32),
            pltpu.VMEM((RING, CK, H), jnp.float32),
        ] + _SEM_RING,
    )
    return f(p, q, src, dst)


def _edge_out_body(pq_ref, eb1, w2p8, b2p8, o_ref):
    blk = pq_ref[...]
    eh = jnp.maximum(blk[:, :H] + blk[:, H:] + eb1[...], 0.0)
    logits = lax.dot_general(
        w2p8[...], eh.astype(jnp.bfloat16), (((1,), (1,)), ((), ())),
        preferred_element_type=jnp.float32) + b2p8[...]
    mask = lax.broadcasted_iota(jnp.int32, logits.shape, 0) < 3
    lm = jnp.where(mask, logits, -1e30)
    m = jnp.max(lm, axis=0, keepdims=True)
    ex = jnp.where(mask, jnp.exp(logits - m), 0.0)
    lse = jnp.log(jnp.sum(ex, axis=0, keepdims=True))
    o_ref[...] = logits - m - lse


def _make_edge_out(ne, bm=6400):
    return pl.pallas_call(
        _edge_out_body,
        grid=(ne // bm,),
        in_specs=[
            pl.BlockSpec((bm, 2 * H), lambda i: (i, 0)),
            pl.BlockSpec((1, H), lambda i: (0, 0)),
            pl.BlockSpec((8, H), lambda i: (0, 0)),
            pl.BlockSpec((8, 1), lambda i: (0, 0)),
        ],
        out_specs=pl.BlockSpec((8, bm), lambda i: (0, i)),
        out_shape=jax.ShapeDtypeStruct((8, ne), jnp.float32),
    )


_edge_out_a = _make_edge_out(EA)
_edge_out_b = _make_edge_out(EB)


def _rgcn_layer(x_in, w_rel, w_root, mm4, gidx, sidx, zrows,
                zcnt=None, ones_ck=None, counts=None):
    w_all = jnp.concatenate([w_root[None], w_rel],
                            axis=0).astype(jnp.bfloat16)
    root, ytab = mm4(x_in, w_all)
    if counts is None:
        acc, cnt = _sc_scatter_counts(ytab, gidx, sidx, zrows, zcnt, ones_ck)
        counts = (cnt[0].reshape(R, NP, 1), cnt[1].reshape(R, NP, 1))
    else:
        acc = _sc_scatter(ytab, gidx, sidx, zrows)
    acc_a = acc[0].reshape(R, NP, H)
    acc_b = acc[1].reshape(R, NP, H)
    ca, cb = counts
    parts = ([root] + [acc_a[r] for r in range(R)] + [acc_b[r] for r in range(R)]
             + [ca[r] for r in range(R)] + [cb[r] for r in range(R)])
    return parts, counts


def kernel(x, edge_index, edge_type, w1_rel, w1_root, b1, w2_rel, w2_root, b2,
           w3_rel, w3_root, b3, ln_g, ln_b, ec_w1, ec_b1, ec_w2, ec_b2,
           nc_w1, nc_b1, nc_w2, nc_b2):
    src = edge_index[0]
    dst = edge_index[1]
    g2, s2 = _idx_call(edge_type.reshape(E // 128, 128),
                       src.reshape(E // 128, 128),
                       dst.reshape(E // 128, 128))
    gidx = g2.reshape(E)
    sidx = s2.reshape(E)

    zrows = jnp.zeros((ROWS_T, H), jnp.bfloat16)
    zcnt = jnp.zeros((CNT_T,), jnp.float32)
    ones_ck = jnp.ones((CKS,), jnp.float32)

    parts, counts = _rgcn_layer(x, w1_rel, w1_root, _mm4_din, gidx, sidx,
                                zrows, zcnt=zcnt, ones_ck=ones_ck)
    x1 = _combine(*parts, b1.reshape(1, H))

    parts, _ = _rgcn_layer(x1, w2_rel, w2_root, _mm4_h, gidx, sidx, zrows,
                           counts=counts)
    x2 = _combine(*parts, b2.reshape(1, H))

    parts, _ = _rgcn_layer(x2, w3_rel, w3_root, _mm4_h, gidx, sidx, zrows,
                           counts=counts)
    ec_w1t = ec_w1.T
    p, q, node_out = _combine3(
        *parts, b3.reshape(1, H), ln_g.reshape(1, H), ln_b.reshape(1, H),
        ec_w1t[:H], ec_w1t[H:], nc_w1.T, nc_b1.reshape(1, H // 2),
        nc_w2.T, nc_b2.reshape(1, 2))

    w2p8 = jnp.pad(ec_w2, ((0, 5), (0, 0))).astype(jnp.bfloat16)
    b2p8 = jnp.pad(ec_b2, (0, 5)).reshape(8, 1)
    eb1 = ec_b1.reshape(1, H)
    ehpq_a = _sc_edge_gather(p, q, src[:EA], dst[:EA], EA)
    ehpq_b = _sc_edge_gather(p, q, src[EA:], dst[EA:], EB)
    lt_a = _edge_out_a(ehpq_a, eb1, w2p8, b2p8)
    lt_b = _edge_out_b(ehpq_b, eb1, w2p8, b2p8)
    edge_out = jnp.concatenate([lt_a[:3].T, lt_b[:3].T], axis=0)

    return (edge_out, node_out)


# speedup vs baseline: 1.1089x; 1.0291x over previous
import functools

import jax
import jax.numpy as jnp
from jax import lax
from jax.experimental import pallas as pl
from jax.experimental.pallas import tpu as pltpu
from jax.experimental.pallas import tpu_sc as plsc

N = 10000
NP = 10240
E = 320000
DIN = 128
H = 64
R = 3

NC = 2
NS = 16
NW = NC * NS
EW = E // NW
CK = 80
NCHUNK = EW // CK
CKS = 80
NCHUNKS = EW // CKS
ROWS = R * N
ROWS_PAD = R * NP
ROWS_T = ROWS_PAD // NS
CNT_PAD = 30720
CNT_T = CNT_PAD // NS

_MESH = plsc.VectorSubcoreMesh(core_axis_name="core", subcore_axis_name="subcore")
_SC_PARAMS = pltpu.CompilerParams(use_tc_tiling_on_sc=False)


def _idx_body(et_ref, src_ref, dst_ref, g_ref, s_ref):
    et = et_ref[...]
    g_ref[...] = et * N + src_ref[...]
    s_ref[...] = et * NP + dst_ref[...]


_idx_call = pl.pallas_call(
    _idx_body,
    out_shape=(
        jax.ShapeDtypeStruct((E // 128, 128), jnp.int32),
        jax.ShapeDtypeStruct((E // 128, 128), jnp.int32),
    ),
)


BM = 2000
NB = N // BM


def _mm4_body(x_ref, w_ref, root_ref, ytab_ref):
    m = jnp.dot(x_ref[...].astype(jnp.bfloat16), w_ref[0],
                preferred_element_type=jnp.float32)
    root_ref[...] = m
    ytab_ref[...] = m.astype(jnp.bfloat16)


def _make_mm4(din):
    return pl.pallas_call(
        _mm4_body,
        grid=(NB, R + 1),
        in_specs=[
            pl.BlockSpec((BM, din), lambda i, r: (i, 0)),
            pl.BlockSpec((1, din, H), lambda i, r: (r, 0, 0)),
        ],
        out_specs=(
            pl.BlockSpec((BM, H), lambda i, r: (jnp.where(r == 0, i, NB), 0)),
            pl.BlockSpec((BM, H),
                         lambda i, r: (jnp.where(r == 0, R * NB,
                                                 (r - 1) * NB + i), 0)),
        ),
        out_shape=(
            jax.ShapeDtypeStruct((N + BM, H), jnp.float32),
            jax.ShapeDtypeStruct((ROWS + BM, H), jnp.bfloat16),
        ),
    )


_mm4_din = _make_mm4(DIN)
_mm4_h = _make_mm4(H)


RING = 5
MAIN_ROUNDS = NCHUNK // RING
RINGS = 5
ROUNDS_S = NCHUNKS // RINGS


def _scatter_body(with_counts):
    def body(ytab_h, gidx_h, sidx_h, zrows_h, *rest):
        if with_counts:
            zcnt_h, ones_h, acc_out, cnt_out, gbuf, sbuf, msg, ones_v, \
                acc_sh, cnt_sh = rest[:10]
            sems = rest[10:]
        else:
            acc_out, gbuf, sbuf, msg, acc_sh = rest[:5]
            sems = rest[5:]
        gsem = sems[:RINGS]
        ssem = sems[RINGS:2 * RINGS]
        c = lax.axis_index("core")
        s = lax.axis_index("subcore")
        wid = c * NS + s
        pltpu.sync_copy(zrows_h, acc_sh.at[pl.ds(s * ROWS_T, ROWS_T)])
        if with_counts:
            pltpu.sync_copy(zcnt_h, cnt_sh.at[pl.ds(s * CNT_T, CNT_T)])
            pltpu.sync_copy(ones_h, ones_v)
        plsc.subcore_barrier()

        def load_and_gather(b, ch):
            base = wid * EW + ch * CKS
            pltpu.sync_copy(gidx_h.at[pl.ds(base, CKS)], gbuf.at[b])
            pltpu.sync_copy(sidx_h.at[pl.ds(base, CKS)], sbuf.at[b])
            pltpu.async_copy(ytab_h.at[gbuf.at[b]], msg.at[b], gsem[b])

        def gather_wait(b):
            pltpu.make_async_copy(ytab_h.at[gbuf.at[b]], msg.at[b],
                                  gsem[b]).wait()

        def scatter_wait(b):
            pltpu.make_async_copy(msg.at[b], acc_sh.at[sbuf.at[b]],
                                  ssem[b]).wait()

        for b in range(RINGS):
            load_and_gather(b, b)

        @pl.loop(0, ROUNDS_S)
        def _(k):
            for b in range(RINGS):
                ch = RINGS * k + b
                gather_wait(b)
                pltpu.async_copy(msg.at[b], acc_sh.at[sbuf.at[b]], ssem[b],
                                 add=True)
                if with_counts:
                    pltpu.sync_copy(ones_v, cnt_sh.at[sbuf.at[b]], add=True)

                @pl.when(k < ROUNDS_S - 1)
                def _():
                    scatter_wait(b)
                    load_and_gather(b, ch + RINGS)

        for b in range(RINGS):
            scatter_wait(b)

        plsc.subcore_barrier()
        pltpu.sync_copy(acc_sh.at[pl.ds(s * ROWS_T, ROWS_T)],
                        acc_out.at[c, pl.ds(s * ROWS_T, ROWS_T)])
        if with_counts:
            pltpu.sync_copy(cnt_sh.at[pl.ds(s * CNT_T, CNT_T)],
                            cnt_out.at[c, pl.ds(s * CNT_T, CNT_T)])

    return body


_SEM_RING = [pltpu.SemaphoreType.DMA] * (2 * RING)
_SEM_RING_S = [pltpu.SemaphoreType.DMA] * (2 * RINGS)


def _sc_scatter_counts(ytab, gidx, sidx, zrows, zcnt, ones_ck):
    f = pl.kernel(
        _scatter_body(True),
        out_type=(
            jax.ShapeDtypeStruct((NC, ROWS_PAD, H), jnp.bfloat16),
            jax.ShapeDtypeStruct((NC, CNT_PAD), jnp.float32),
        ),
        mesh=_MESH,
        compiler_params=_SC_PARAMS,
        scratch_types=[
            pltpu.VMEM((RINGS, CKS), jnp.int32),
            pltpu.VMEM((RINGS, CKS), jnp.int32),
            pltpu.VMEM((RINGS, CKS, H), jnp.bfloat16),
            pltpu.VMEM((CKS,), jnp.float32),
            pltpu.VMEM_SHARED((ROWS_PAD, H), jnp.bfloat16),
            pltpu.VMEM_SHARED((CNT_PAD,), jnp.float32),
        ] + _SEM_RING_S,
    )
    return f(ytab, gidx, sidx, zrows, zcnt, ones_ck)


def _sc_scatter(ytab, gidx, sidx, zrows):
    f = pl.kernel(
        _scatter_body(False),
        out_type=jax.ShapeDtypeStruct((NC, ROWS_PAD, H), jnp.bfloat16),
        mesh=_MESH,
        compiler_params=_SC_PARAMS,
        scratch_types=[
            pltpu.VMEM((RINGS, CKS), jnp.int32),
            pltpu.VMEM((RINGS, CKS), jnp.int32),
            pltpu.VMEM((RINGS, CKS, H), jnp.bfloat16),
            pltpu.VMEM_SHARED((ROWS_PAD, H), jnp.bfloat16),
        ] + _SEM_RING_S,
    )
    return f(ytab, gidx, sidx, zrows)


def _combine_body(root, a0, a1, a2, b0, b1, b2, c0, c1, c2, d0, d1, d2,
                  bias, o):
    h = root[...] + bias[...]
    for aa, bb, cc, dd in ((a0, b0, c0, d0), (a1, b1, c1, d1),
                           (a2, b2, c2, d2)):
        cnt = jnp.maximum(cc[...] + dd[...], 1.0)
        h = h + (aa[...].astype(jnp.float32)
                 + bb[...].astype(jnp.float32)) / cnt
    o[...] = jnp.maximum(h, 0.0)


def _make_combine(bm=1000):
    nh_spec = pl.BlockSpec((bm, H), lambda i: (i, 0))
    n1_spec = pl.BlockSpec((bm, 1), lambda i: (i, 0))
    b_spec = pl.BlockSpec((1, H), lambda i: (0, 0))
    return pl.pallas_call(
        _combine_body,
        grid=(N // bm,),
        in_specs=[nh_spec] * 7 + [n1_spec] * 6 + [b_spec],
        out_specs=nh_spec,
        out_shape=jax.ShapeDtypeStruct((N, H), jnp.float32),
    )


_combine = _make_combine()


def _combine3_body(root, a0, a1, a2, b0, b1, b2, c0, c1, c2, d0, d1, d2,
                   bias, g, bln, wa, wb, nw1, nb1, nw2, nb2,
                   p_o, q_o, node_o):
    h = root[...] + bias[...]
    for aa, bb, cc, dd in ((a0, b0, c0, d0), (a1, b1, c1, d1),
                           (a2, b2, c2, d2)):
        cnt = jnp.maximum(cc[...] + dd[...], 1.0)
        h = h + (aa[...].astype(jnp.float32)
                 + bb[...].astype(jnp.float32)) / cnt
    mu = jnp.mean(h, axis=-1, keepdims=True)
    var = jnp.mean((h - mu) ** 2, axis=-1, keepdims=True)
    xn = g[...] * (h - mu) / jnp.sqrt(var + 1e-5) + bln[...]
    p_o[...] = jnp.dot(xn, wa[...], preferred_element_type=jnp.float32)
    q_o[...] = jnp.dot(xn, wb[...], preferred_element_type=jnp.float32)
    nh = jnp.maximum(
        jnp.dot(xn, nw1[...], preferred_element_type=jnp.float32) + nb1[...],
        0.0)
    lg = jnp.dot(nh, nw2[...], preferred_element_type=jnp.float32) + nb2[...]
    m = jnp.max(lg, axis=-1, keepdims=True)
    l = lg - m
    node_o[...] = l - jnp.log(jnp.sum(jnp.exp(l), axis=-1, keepdims=True))


def _make_combine3(bm=1000):
    nh_spec = pl.BlockSpec((bm, H), lambda i: (i, 0))
    n1_spec = pl.BlockSpec((bm, 1), lambda i: (i, 0))
    b_spec = pl.BlockSpec((1, H), lambda i: (0, 0))
    return pl.pallas_call(
        _combine3_body,
        grid=(N // bm,),
        in_specs=(
            [nh_spec] * 7 + [n1_spec] * 6 + [b_spec] * 3
            + [pl.BlockSpec((H, H), lambda i: (0, 0))] * 2
            + [pl.BlockSpec((H, H // 2), lambda i: (0, 0)),
               pl.BlockSpec((1, H // 2), lambda i: (0, 0)),
               pl.BlockSpec((H // 2, 2), lambda i: (0, 0)),
               pl.BlockSpec((1, 2), lambda i: (0, 0))]
        ),
        out_specs=(
            nh_spec,
            nh_spec,
            pl.BlockSpec((bm, 2), lambda i: (i, 0)),
        ),
        out_shape=(
            jax.ShapeDtypeStruct((N, H), jnp.float32),
            jax.ShapeDtypeStruct((N, H), jnp.float32),
            jax.ShapeDtypeStruct((N, 2), jnp.float32),
        ),
    )


_combine3 = _make_combine3()


EA = 192000
EB = E - EA


def _sc_edge_gather(p, q, src, dst, ne):
    ew2 = ne // NW
    nchunk2 = ew2 // CK
    rounds_e = nchunk2 // RING
    def body(p_h, q_h, src_h, dst_h, ehpq_out, sibuf, dibuf, bp, bq, *sems):
        gsem = sems[:RING]
        wsem = sems[RING:2 * RING]
        c = lax.axis_index("core")
        s = lax.axis_index("subcore")
        wid = c * NS + s

        def load_and_gather(b, ch):
            base = wid * ew2 + ch * CK
            pltpu.sync_copy(src_h.at[pl.ds(base, CK)], sibuf.at[b])
            pltpu.sync_copy(dst_h.at[pl.ds(base, CK)], dibuf.at[b])
            pltpu.async_copy(p_h.at[sibuf.at[b]], bp.at[b], gsem[b])
            pltpu.async_copy(q_h.at[dibuf.at[b]], bq.at[b], gsem[b])

        def gather_wait(b):
            pltpu.make_async_copy(p_h.at[sibuf.at[b]], bp.at[b],
                                  gsem[b]).wait()
            pltpu.make_async_copy(q_h.at[dibuf.at[b]], bq.at[b],
                                  gsem[b]).wait()

        def write_start(b, ch):
            base = wid * ew2 + ch * CK
            pltpu.async_copy(bp.at[b],
                             ehpq_out.at[pl.ds(base, CK), pl.ds(0, H)],
                             wsem[b])
            pltpu.async_copy(bq.at[b],
                             ehpq_out.at[pl.ds(base, CK), pl.ds(H, H)],
                             wsem[b])

        def write_wait(b, ch):
            base = wid * ew2 + ch * CK
            pltpu.make_async_copy(bp.at[b],
                                  ehpq_out.at[pl.ds(base, CK), pl.ds(0, H)],
                                  wsem[b]).wait()
            pltpu.make_async_copy(bq.at[b],
                                  ehpq_out.at[pl.ds(base, CK), pl.ds(H, H)],
                                  wsem[b]).wait()

        for b in range(RING):
            load_and_gather(b, b)

        @pl.loop(0, rounds_e)
        def _(k):
            for b in range(RING):
                ch = RING * k + b
                gather_wait(b)
                write_start(b, ch)

                @pl.when(k < rounds_e - 1)
                def _():
                    write_wait(b, ch)
                    load_and_gather(b, ch + RING)

        for b in range(RING):
            write_wait(b, 0)

    f = pl.kernel(
        body,
        out_type=jax.ShapeDtypeStruct((ne, 2 * H), jnp.float32),
        mesh=_MESH,
        compiler_params=_SC_PARAMS,
        scratch_types=[
            pltpu.VMEM((RING, CK), jnp.int32),
            pltpu.VMEM((RING, CK), jnp.int32),
            pltpu.VMEM((RING, CK, H), jnp.float32),
            pltpu.VMEM((RING, CK, H), jnp.float32),
        ] + _SEM_RING,
    )
    return f(p, q, src, dst)


def _edge_out_body(pq_ref, eb1, w2p8, b2p8, o_ref):
    blk = pq_ref[...]
    eh = jnp.maximum(blk[:, :H] + blk[:, H:] + eb1[...], 0.0)
    logits = lax.dot_general(
        w2p8[...], eh.astype(jnp.bfloat16), (((1,), (1,)), ((), ())),
        preferred_element_type=jnp.float32) + b2p8[...]
    mask = lax.broadcasted_iota(jnp.int32, logits.shape, 0) < 3
    lm = jnp.where(mask, logits, -1e30)
    m = jnp.max(lm, axis=0, keepdims=True)
    ex = jnp.where(mask, jnp.exp(logits - m), 0.0)
    lse = jnp.log(jnp.sum(ex, axis=0, keepdims=True))
    o_ref[...] = logits - m - lse


def _make_edge_out(ne, bm=6400):
    return pl.pallas_call(
        _edge_out_body,
        grid=(ne // bm,),
        in_specs=[
            pl.BlockSpec((bm, 2 * H), lambda i: (i, 0)),
            pl.BlockSpec((1, H), lambda i: (0, 0)),
            pl.BlockSpec((8, H), lambda i: (0, 0)),
            pl.BlockSpec((8, 1), lambda i: (0, 0)),
        ],
        out_specs=pl.BlockSpec((8, bm), lambda i: (0, i)),
        out_shape=jax.ShapeDtypeStruct((8, ne), jnp.float32),
    )


_edge_out_a = _make_edge_out(EA)
_edge_out_b = _make_edge_out(EB)


def _rgcn_layer(x_in, w_rel, w_root, mm4, gidx, sidx, zrows,
                zcnt=None, ones_ck=None, counts=None):
    w_all = jnp.concatenate([w_root[None], w_rel],
                            axis=0).astype(jnp.bfloat16)
    root, ytab = mm4(x_in, w_all)
    if counts is None:
        acc, cnt = _sc_scatter_counts(ytab, gidx, sidx, zrows, zcnt, ones_ck)
        counts = (cnt[0].reshape(R, NP, 1), cnt[1].reshape(R, NP, 1))
    else:
        acc = _sc_scatter(ytab, gidx, sidx, zrows)
    acc_a = acc[0].reshape(R, NP, H)
    acc_b = acc[1].reshape(R, NP, H)
    ca, cb = counts
    parts = ([root] + [acc_a[r] for r in range(R)] + [acc_b[r] for r in range(R)]
             + [ca[r] for r in range(R)] + [cb[r] for r in range(R)])
    return parts, counts


def kernel(x, edge_index, edge_type, w1_rel, w1_root, b1, w2_rel, w2_root, b2,
           w3_rel, w3_root, b3, ln_g, ln_b, ec_w1, ec_b1, ec_w2, ec_b2,
           nc_w1, nc_b1, nc_w2, nc_b2):
    src = edge_index[0]
    dst = edge_index[1]
    g2, s2 = _idx_call(edge_type.reshape(E // 128, 128),
                       src.reshape(E // 128, 128),
                       dst.reshape(E // 128, 128))
    gidx = g2.reshape(E)
    sidx = s2.reshape(E)

    zrows = jnp.zeros((ROWS_T, H), jnp.bfloat16)
    zcnt = jnp.zeros((CNT_T,), jnp.float32)
    ones_ck = jnp.ones((CKS,), jnp.float32)

    parts, counts = _rgcn_layer(x, w1_rel, w1_root, _mm4_din, gidx, sidx,
                                zrows, zcnt=zcnt, ones_ck=ones_ck)
    x1 = _combine(*parts, b1.reshape(1, H))

    parts, _ = _rgcn_layer(x1, w2_rel, w2_root, _mm4_h, gidx, sidx, zrows,
                           counts=counts)
    x2 = _combine(*parts, b2.reshape(1, H))

    parts, _ = _rgcn_layer(x2, w3_rel, w3_root, _mm4_h, gidx, sidx, zrows,
                           counts=counts)
    ec_w1t = ec_w1.T
    p, q, node_out = _combine3(
        *parts, b3.reshape(1, H), ln_g.reshape(1, H), ln_b.reshape(1, H),
        ec_w1t[:H], ec_w1t[H:], nc_w1.T, nc_b1.reshape(1, H // 2),
        nc_w2.T, nc_b2.reshape(1, 2))

    w2p8 = jnp.pad(ec_w2, ((0, 5), (0, 0))).astype(jnp.bfloat16)
    b2p8 = jnp.pad(ec_b2, (0, 5)).reshape(8, 1)
    eb1 = ec_b1.reshape(1, H)
    ehpq_a = _sc_edge_gather(p, q, src[:EA], dst[:EA], EA)
    ehpq_b = _sc_edge_gather(p, q, src[EA:], dst[EA:], EB)
    lt_a = _edge_out_a(ehpq_a, eb1, w2p8, b2p8)
    lt_b = _edge_out_b(ehpq_b, eb1, w2p8, b2p8)
    edge_out = jnp.concatenate([lt_a[:3].T, lt_b[:3].T], axis=0)

    return (edge_out, node_out)


# speedup vs baseline: 1.1143x; 1.0049x over previous
import functools

import jax
import jax.numpy as jnp
from jax import lax
from jax.experimental import pallas as pl
from jax.experimental.pallas import tpu as pltpu
from jax.experimental.pallas import tpu_sc as plsc

N = 10000
NP = 10240
E = 320000
DIN = 128
H = 64
R = 3

NC = 2
NS = 16
NW = NC * NS
EW = E // NW
CK = 80
NCHUNK = EW // CK
CKS = 80
NCHUNKS = EW // CKS
ROWS = R * N
ROWS_PAD = R * NP
ROWS_T = ROWS_PAD // NS
CNT_PAD = 30720
CNT_T = CNT_PAD // NS

_MESH = plsc.VectorSubcoreMesh(core_axis_name="core", subcore_axis_name="subcore")
_SC_PARAMS = pltpu.CompilerParams(use_tc_tiling_on_sc=False)


def _idx_body(et_ref, src_ref, dst_ref, g_ref, s_ref):
    et = et_ref[...]
    g_ref[...] = et * N + src_ref[...]
    s_ref[...] = et * NP + dst_ref[...]


_idx_call = pl.pallas_call(
    _idx_body,
    out_shape=(
        jax.ShapeDtypeStruct((E // 128, 128), jnp.int32),
        jax.ShapeDtypeStruct((E // 128, 128), jnp.int32),
    ),
)


BM = 2000
NB = N // BM


def _mm4_body(x_ref, w_ref, root_ref, ytab_ref):
    m = jnp.dot(x_ref[...].astype(jnp.bfloat16), w_ref[0],
                preferred_element_type=jnp.float32)
    root_ref[...] = m
    ytab_ref[...] = m.astype(jnp.bfloat16)


def _make_mm4(din):
    return pl.pallas_call(
        _mm4_body,
        grid=(NB, R + 1),
        in_specs=[
            pl.BlockSpec((BM, din), lambda i, r: (i, 0)),
            pl.BlockSpec((1, din, H), lambda i, r: (r, 0, 0)),
        ],
        out_specs=(
            pl.BlockSpec((BM, H), lambda i, r: (jnp.where(r == 0, i, NB), 0)),
            pl.BlockSpec((BM, H),
                         lambda i, r: (jnp.where(r == 0, R * NB,
                                                 (r - 1) * NB + i), 0)),
        ),
        out_shape=(
            jax.ShapeDtypeStruct((N + BM, H), jnp.float32),
            jax.ShapeDtypeStruct((ROWS + BM, H), jnp.bfloat16),
        ),
    )


_mm4_din = _make_mm4(DIN)
_mm4_h = _make_mm4(H)


RING = 5
MAIN_ROUNDS = NCHUNK // RING
RINGS = 5
ROUNDS_S = NCHUNKS // RINGS


def _scatter_body(with_counts):
    def body(ytab_h, gidx_h, sidx_h, zrows_h, *rest):
        if with_counts:
            zcnt_h, ones_h, acc_out, cnt_out, gbuf, sbuf, msg, ones_v, \
                acc_sh, cnt_sh = rest[:10]
            sems = rest[10:]
        else:
            acc_out, gbuf, sbuf, msg, acc_sh = rest[:5]
            sems = rest[5:]
        gsem = sems[:RINGS]
        ssem = sems[RINGS:2 * RINGS]
        c = lax.axis_index("core")
        s = lax.axis_index("subcore")
        wid = c * NS + s
        pltpu.sync_copy(zrows_h, acc_sh.at[pl.ds(s * ROWS_T, ROWS_T)])
        if with_counts:
            pltpu.sync_copy(zcnt_h, cnt_sh.at[pl.ds(s * CNT_T, CNT_T)])
            pltpu.sync_copy(ones_h, ones_v)
        plsc.subcore_barrier()

        def load_and_gather(b, ch):
            base = wid * EW + ch * CKS
            pltpu.sync_copy(gidx_h.at[pl.ds(base, CKS)], gbuf.at[b])
            pltpu.sync_copy(sidx_h.at[pl.ds(base, CKS)], sbuf.at[b])
            pltpu.async_copy(ytab_h.at[gbuf.at[b]], msg.at[b], gsem[b])

        def gather_wait(b):
            pltpu.make_async_copy(ytab_h.at[gbuf.at[b]], msg.at[b],
                                  gsem[b]).wait()

        def scatter_wait(b):
            pltpu.make_async_copy(msg.at[b], acc_sh.at[sbuf.at[b]],
                                  ssem[b]).wait()

        for b in range(RINGS):
            load_and_gather(b, b)

        @pl.loop(0, ROUNDS_S)
        def _(k):
            for b in range(RINGS):
                ch = RINGS * k + b
                gather_wait(b)
                pltpu.async_copy(msg.at[b], acc_sh.at[sbuf.at[b]], ssem[b],
                                 add=True)
                if with_counts:
                    pltpu.sync_copy(ones_v, cnt_sh.at[sbuf.at[b]], add=True)

                @pl.when(k < ROUNDS_S - 1)
                def _():
                    scatter_wait(b)
                    load_and_gather(b, ch + RINGS)

        for b in range(RINGS):
            scatter_wait(b)

        plsc.subcore_barrier()
        pltpu.sync_copy(acc_sh.at[pl.ds(s * ROWS_T, ROWS_T)],
                        acc_out.at[c, pl.ds(s * ROWS_T, ROWS_T)])
        if with_counts:
            pltpu.sync_copy(cnt_sh.at[pl.ds(s * CNT_T, CNT_T)],
                            cnt_out.at[c, pl.ds(s * CNT_T, CNT_T)])

    return body


_SEM_RING = [pltpu.SemaphoreType.DMA] * (2 * RING)
_SEM_RING_S = [pltpu.SemaphoreType.DMA] * (2 * RINGS)


def _sc_scatter_counts(ytab, gidx, sidx, zrows, zcnt, ones_ck):
    f = pl.kernel(
        _scatter_body(True),
        out_type=(
            jax.ShapeDtypeStruct((NC, ROWS_PAD, H), jnp.bfloat16),
            jax.ShapeDtypeStruct((NC, CNT_PAD), jnp.float32),
        ),
        mesh=_MESH,
        compiler_params=_SC_PARAMS,
        scratch_types=[
            pltpu.VMEM((RINGS, CKS), jnp.int32),
            pltpu.VMEM((RINGS, CKS), jnp.int32),
            pltpu.VMEM((RINGS, CKS, H), jnp.bfloat16),
            pltpu.VMEM((CKS,), jnp.float32),
            pltpu.VMEM_SHARED((ROWS_PAD, H), jnp.bfloat16),
            pltpu.VMEM_SHARED((CNT_PAD,), jnp.float32),
        ] + _SEM_RING_S,
    )
    return f(ytab, gidx, sidx, zrows, zcnt, ones_ck)


def _sc_scatter(ytab, gidx, sidx, zrows):
    f = pl.kernel(
        _scatter_body(False),
        out_type=jax.ShapeDtypeStruct((NC, ROWS_PAD, H), jnp.bfloat16),
        mesh=_MESH,
        compiler_params=_SC_PARAMS,
        scratch_types=[
            pltpu.VMEM((RINGS, CKS), jnp.int32),
            pltpu.VMEM((RINGS, CKS), jnp.int32),
            pltpu.VMEM((RINGS, CKS, H), jnp.bfloat16),
            pltpu.VMEM_SHARED((ROWS_PAD, H), jnp.bfloat16),
        ] + _SEM_RING_S,
    )
    return f(ytab, gidx, sidx, zrows)


def _combine_body(root, a0, a1, a2, b0, b1, b2, c0, c1, c2, d0, d1, d2,
                  bias, o):
    h = root[...] + bias[...]
    for aa, bb, cc, dd in ((a0, b0, c0, d0), (a1, b1, c1, d1),
                           (a2, b2, c2, d2)):
        cnt = jnp.maximum(cc[...] + dd[...], 1.0)
        h = h + (aa[...].astype(jnp.float32)
                 + bb[...].astype(jnp.float32)) / cnt
    o[...] = jnp.maximum(h, 0.0)


def _make_combine(bm=2000):
    nh_spec = pl.BlockSpec((bm, H), lambda i: (i, 0))
    n1_spec = pl.BlockSpec((bm, 1), lambda i: (i, 0))
    b_spec = pl.BlockSpec((1, H), lambda i: (0, 0))
    return pl.pallas_call(
        _combine_body,
        grid=(N // bm,),
        in_specs=[nh_spec] * 7 + [n1_spec] * 6 + [b_spec],
        out_specs=nh_spec,
        out_shape=jax.ShapeDtypeStruct((N, H), jnp.float32),
    )


_combine = _make_combine()


def _combine3_body(root, a0, a1, a2, b0, b1, b2, c0, c1, c2, d0, d1, d2,
                   bias, g, bln, wa, wb, nw1, nb1, nw2, nb2,
                   p_o, q_o, node_o):
    h = root[...] + bias[...]
    for aa, bb, cc, dd in ((a0, b0, c0, d0), (a1, b1, c1, d1),
                           (a2, b2, c2, d2)):
        cnt = jnp.maximum(cc[...] + dd[...], 1.0)
        h = h + (aa[...].astype(jnp.float32)
                 + bb[...].astype(jnp.float32)) / cnt
    mu = jnp.mean(h, axis=-1, keepdims=True)
    var = jnp.mean((h - mu) ** 2, axis=-1, keepdims=True)
    xn = g[...] * (h - mu) / jnp.sqrt(var + 1e-5) + bln[...]
    p_o[...] = jnp.dot(xn, wa[...], preferred_element_type=jnp.float32)
    q_o[...] = jnp.dot(xn, wb[...], preferred_element_type=jnp.float32)
    nh = jnp.maximum(
        jnp.dot(xn, nw1[...], preferred_element_type=jnp.float32) + nb1[...],
        0.0)
    lg = jnp.dot(nh, nw2[...], preferred_element_type=jnp.float32) + nb2[...]
    m = jnp.max(lg, axis=-1, keepdims=True)
    l = lg - m
    node_o[...] = l - jnp.log(jnp.sum(jnp.exp(l), axis=-1, keepdims=True))


def _make_combine3(bm=2000):
    nh_spec = pl.BlockSpec((bm, H), lambda i: (i, 0))
    n1_spec = pl.BlockSpec((bm, 1), lambda i: (i, 0))
    b_spec = pl.BlockSpec((1, H), lambda i: (0, 0))
    return pl.pallas_call(
        _combine3_body,
        grid=(N // bm,),
        in_specs=(
            [nh_spec] * 7 + [n1_spec] * 6 + [b_spec] * 3
            + [pl.BlockSpec((H, H), lambda i: (0, 0))] * 2
            + [pl.BlockSpec((H, H // 2), lambda i: (0, 0)),
               pl.BlockSpec((1, H // 2), lambda i: (0, 0)),
               pl.BlockSpec((H // 2, 2), lambda i: (0, 0)),
               pl.BlockSpec((1, 2), lambda i: (0, 0))]
        ),
        out_specs=(
            nh_spec,
            nh_spec,
            pl.BlockSpec((bm, 2), lambda i: (i, 0)),
        ),
        out_shape=(
            jax.ShapeDtypeStruct((N, H), jnp.float32),
            jax.ShapeDtypeStruct((N, H), jnp.float32),
            jax.ShapeDtypeStruct((N, 2), jnp.float32),
        ),
    )


_combine3 = _make_combine3()


EA = 192000
EB = E - EA


def _sc_edge_gather(p, q, src, dst, ne):
    ew2 = ne // NW
    nchunk2 = ew2 // CK
    rounds_e = nchunk2 // RING
    def body(p_h, q_h, src_h, dst_h, ehpq_out, sibuf, dibuf, bp, bq, *sems):
        gsem = sems[:RING]
        wsem = sems[RING:2 * RING]
        c = lax.axis_index("core")
        s = lax.axis_index("subcore")
        wid = c * NS + s

        def load_and_gather(b, ch):
            base = wid * ew2 + ch * CK
            pltpu.sync_copy(src_h.at[pl.ds(base, CK)], sibuf.at[b])
            pltpu.sync_copy(dst_h.at[pl.ds(base, CK)], dibuf.at[b])
            pltpu.async_copy(p_h.at[sibuf.at[b]], bp.at[b], gsem[b])
            pltpu.async_copy(q_h.at[dibuf.at[b]], bq.at[b], gsem[b])

        def gather_wait(b):
            pltpu.make_async_copy(p_h.at[sibuf.at[b]], bp.at[b],
                                  gsem[b]).wait()
            pltpu.make_async_copy(q_h.at[dibuf.at[b]], bq.at[b],
                                  gsem[b]).wait()

        def write_start(b, ch):
            base = wid * ew2 + ch * CK
            pltpu.async_copy(bp.at[b],
                             ehpq_out.at[pl.ds(base, CK), pl.ds(0, H)],
                             wsem[b])
            pltpu.async_copy(bq.at[b],
                             ehpq_out.at[pl.ds(base, CK), pl.ds(H, H)],
                             wsem[b])

        def write_wait(b, ch):
            base = wid * ew2 + ch * CK
            pltpu.make_async_copy(bp.at[b],
                                  ehpq_out.at[pl.ds(base, CK), pl.ds(0, H)],
                                  wsem[b]).wait()
            pltpu.make_async_copy(bq.at[b],
                                  ehpq_out.at[pl.ds(base, CK), pl.ds(H, H)],
                                  wsem[b]).wait()

        for b in range(RING):
            load_and_gather(b, b)

        @pl.loop(0, rounds_e)
        def _(k):
            for b in range(RING):
                ch = RING * k + b
                gather_wait(b)
                write_start(b, ch)

                @pl.when(k < rounds_e - 1)
                def _():
                    write_wait(b, ch)
                    load_and_gather(b, ch + RING)

        for b in range(RING):
            write_wait(b, 0)

    f = pl.kernel(
        body,
        out_type=jax.ShapeDtypeStruct((ne, 2 * H), jnp.float32),
        mesh=_MESH,
        compiler_params=_SC_PARAMS,
        scratch_types=[
            pltpu.VMEM((RING, CK), jnp.int32),
            pltpu.VMEM((RING, CK), jnp.int32),
            pltpu.VMEM((RING, CK, H), jnp.float32),
            pltpu.VMEM((RING, CK, H), jnp.float32),
        ] + _SEM_RING,
    )
    return f(p, q, src, dst)


def _edge_out_body(pq_ref, eb1, w2p8, b2p8, o_ref):
    blk = pq_ref[...]
    eh = jnp.maximum(blk[:, :H] + blk[:, H:] + eb1[...], 0.0)
    logits = lax.dot_general(
        w2p8[...], eh.astype(jnp.bfloat16), (((1,), (1,)), ((), ())),
        preferred_element_type=jnp.float32) + b2p8[...]
    mask = lax.broadcasted_iota(jnp.int32, logits.shape, 0) < 3
    lm = jnp.where(mask, logits, -1e30)
    m = jnp.max(lm, axis=0, keepdims=True)
    ex = jnp.where(mask, jnp.exp(logits - m), 0.0)
    lse = jnp.log(jnp.sum(ex, axis=0, keepdims=True))
    o_ref[...] = logits - m - lse


def _make_edge_out(ne, bm=6400):
    return pl.pallas_call(
        _edge_out_body,
        grid=(ne // bm,),
        in_specs=[
            pl.BlockSpec((bm, 2 * H), lambda i: (i, 0)),
            pl.BlockSpec((1, H), lambda i: (0, 0)),
            pl.BlockSpec((8, H), lambda i: (0, 0)),
            pl.BlockSpec((8, 1), lambda i: (0, 0)),
        ],
        out_specs=pl.BlockSpec((8, bm), lambda i: (0, i)),
        out_shape=jax.ShapeDtypeStruct((8, ne), jnp.float32),
    )


_edge_out_a = _make_edge_out(EA)
_edge_out_b = _make_edge_out(EB)


def _rgcn_layer(x_in, w_rel, w_root, mm4, gidx, sidx, zrows,
                zcnt=None, ones_ck=None, counts=None):
    w_all = jnp.concatenate([w_root[None], w_rel],
                            axis=0).astype(jnp.bfloat16)
    root, ytab = mm4(x_in, w_all)
    if counts is None:
        acc, cnt = _sc_scatter_counts(ytab, gidx, sidx, zrows, zcnt, ones_ck)
        counts = (cnt[0].reshape(R, NP, 1), cnt[1].reshape(R, NP, 1))
    else:
        acc = _sc_scatter(ytab, gidx, sidx, zrows)
    acc_a = acc[0].reshape(R, NP, H)
    acc_b = acc[1].reshape(R, NP, H)
    ca, cb = counts
    parts = ([root] + [acc_a[r] for r in range(R)] + [acc_b[r] for r in range(R)]
             + [ca[r] for r in range(R)] + [cb[r] for r in range(R)])
    return parts, counts


def kernel(x, edge_index, edge_type, w1_rel, w1_root, b1, w2_rel, w2_root, b2,
           w3_rel, w3_root, b3, ln_g, ln_b, ec_w1, ec_b1, ec_w2, ec_b2,
           nc_w1, nc_b1, nc_w2, nc_b2):
    src = edge_index[0]
    dst = edge_index[1]
    g2, s2 = _idx_call(edge_type.reshape(E // 128, 128),
                       src.reshape(E // 128, 128),
                       dst.reshape(E // 128, 128))
    gidx = g2.reshape(E)
    sidx = s2.reshape(E)

    zrows = jnp.zeros((ROWS_T, H), jnp.bfloat16)
    zcnt = jnp.zeros((CNT_T,), jnp.float32)
    ones_ck = jnp.ones((CKS,), jnp.float32)

    parts, counts = _rgcn_layer(x, w1_rel, w1_root, _mm4_din, gidx, sidx,
                                zrows, zcnt=zcnt, ones_ck=ones_ck)
    x1 = _combine(*parts, b1.reshape(1, H))

    parts, _ = _rgcn_layer(x1, w2_rel, w2_root, _mm4_h, gidx, sidx, zrows,
                           counts=counts)
    x2 = _combine(*parts, b2.reshape(1, H))

    parts, _ = _rgcn_layer(x2, w3_rel, w3_root, _mm4_h, gidx, sidx, zrows,
                           counts=counts)
    ec_w1t = ec_w1.T
    p, q, node_out = _combine3(
        *parts, b3.reshape(1, H), ln_g.reshape(1, H), ln_b.reshape(1, H),
        ec_w1t[:H], ec_w1t[H:], nc_w1.T, nc_b1.reshape(1, H // 2),
        nc_w2.T, nc_b2.reshape(1, 2))

    w2p8 = jnp.pad(ec_w2, ((0, 5), (0, 0))).astype(jnp.bfloat16)
    b2p8 = jnp.pad(ec_b2, (0, 5)).reshape(8, 1)
    eb1 = ec_b1.reshape(1, H)
    ehpq_a = _sc_edge_gather(p, q, src[:EA], dst[:EA], EA)
    ehpq_b = _sc_edge_gather(p, q, src[EA:], dst[EA:], EB)
    lt_a = _edge_out_a(ehpq_a, eb1, w2p8, b2p8)
    lt_b = _edge_out_b(ehpq_b, eb1, w2p8, b2p8)
    edge_out = jnp.concatenate([lt_a[:3].T, lt_b[:3].T], axis=0)

    return (edge_out, node_out)
